# Initial kernel scaffold; baseline (speedup 1.0000x reference)
import jax
import jax.numpy as jnp
from jax.experimental import pallas as pl


def kernel(codes, step_mask, codebook, m_acc, n_acc):
    raise NotImplementedError("write your pallas kernel here")


# speedup vs baseline: 1.8483x; 1.8483x over previous
import functools

import jax
import jax.numpy as jnp
from jax.experimental import pallas as pl
from jax.experimental.pallas import tpu as pltpu

NUM_NT = 1024
NT_DIM = 64
DECAY = 0.99

_R = 1024


def _body(flat_ref, mask_ref, cb_ref, macc_ref, nacc_ref,
          quant_ref, push_ref, newm_ref, newn_ref,
          accm_s, accn_s, *, nsteps):
    i = pl.program_id(0)
    x = flat_ref[...]
    cb = cb_ref[...]
    xsq = jnp.sum(x * x, axis=1, keepdims=True)
    cbsq = jnp.sum(cb * cb, axis=1)
    scores = jax.lax.dot_general(x, cb, (((1,), (1,)), ((), ())),
                                 preferred_element_type=jnp.float32)
    d2 = xsq - 2.0 * scores + cbsq[None, :]
    idx = jnp.argmin(d2, axis=1).astype(jnp.int32)
    m = mask_ref[0, 0, :]
    onehot = (idx[:, None] ==
              jax.lax.broadcasted_iota(jnp.int32, (_R, NUM_NT), 1)
              ).astype(jnp.float32)
    quant = jax.lax.dot_general(onehot, cb, (((1,), (0,)), ((), ())),
                                preferred_element_type=jnp.float32)
    quant_ref[...] = quant
    push_ref[0, 0, :] = idx * m.astype(jnp.int32)
    w = onehot * m[:, None]
    contrib_m = jax.lax.dot_general(w, x, (((0,), (0,)), ((), ())),
                                    preferred_element_type=jnp.float32)
    contrib_n = jnp.sum(w, axis=0)[None, :]

    @pl.when(i == 0)
    def _():
        accm_s[...] = contrib_m
        accn_s[...] = contrib_n

    @pl.when(i > 0)
    def _():
        accm_s[...] += contrib_m
        accn_s[...] += contrib_n

    @pl.when(i == nsteps - 1)
    def _():
        newm_ref[...] = DECAY * macc_ref[...] + (1.0 - DECAY) * accm_s[...]
        newn_ref[...] = DECAY * nacc_ref[...] + (1.0 - DECAY) * accn_s[...]


def kernel(codes, step_mask, codebook, m_acc, n_acc):
    B, T, two, D = codes.shape
    K = codebook.shape[0]
    N = B * T * two
    nsteps = N // _R
    flat = codes.reshape(N, D)
    maskf = jnp.broadcast_to(step_mask[:, :, None], (B, T, two)).reshape(
        nsteps, 1, _R)
    nacc2 = n_acc.reshape(1, K)

    grid = (nsteps,)
    out_shapes = (
        jax.ShapeDtypeStruct((N, D), jnp.float32),
        jax.ShapeDtypeStruct((nsteps, 1, _R), jnp.int32),
        jax.ShapeDtypeStruct((K, D), jnp.float32),
        jax.ShapeDtypeStruct((1, K), jnp.float32),
    )
    quant, push3, new_m, new_n2 = pl.pallas_call(
        functools.partial(_body, nsteps=nsteps),
        grid=grid,
        in_specs=[
            pl.BlockSpec((_R, D), lambda i: (i, 0)),
            pl.BlockSpec((1, 1, _R), lambda i: (i, 0, 0)),
            pl.BlockSpec((K, D), lambda i: (0, 0)),
            pl.BlockSpec((K, D), lambda i: (0, 0)),
            pl.BlockSpec((1, K), lambda i: (0, 0)),
        ],
        out_specs=[
            pl.BlockSpec((_R, D), lambda i: (i, 0)),
            pl.BlockSpec((1, 1, _R), lambda i: (i, 0, 0)),
            pl.BlockSpec((K, D), lambda i: (0, 0)),
            pl.BlockSpec((1, K), lambda i: (0, 0)),
        ],
        out_shape=out_shapes,
        scratch_shapes=[
            pltpu.VMEM((K, D), jnp.float32),
            pltpu.VMEM((1, K), jnp.float32),
        ],
    )(flat, maskf, codebook, m_acc, nacc2)
    return (quant.reshape(B, T, two, D),
            push3.reshape(B, T, two),
            new_m,
            new_n2.reshape(K))
